# baseline (speedup 1.0000x reference)
import functools

import jax
import jax.numpy as jnp
from jax.experimental import pallas as pl
from jax.experimental.pallas import tpu as pltpu

_BI = 256
_BJ = 128


def _frd_kernel(batch_sm, rowcode_ref, colcode_ref, z_ref, zflat_ref,
                patt_ref, e_ref, f_ref, seg_ref, out_ref, *, bi, bj, r):
    i0 = pl.program_id(0)
    j0 = pl.program_id(1)

    row_lo = batch_sm[i0 * bi]
    row_hi = batch_sm[i0 * bi + bi - 1]
    col_lo = batch_sm[j0 * bj]
    col_hi = batch_sm[j0 * bj + bj - 1]
    overlap = jnp.logical_and(row_lo <= col_hi, col_lo <= row_hi)

    patt = patt_ref[...]

    @pl.when(jnp.logical_not(overlap))
    def _pattern_only():
        out_ref[...] = jnp.broadcast_to(patt, out_ref.shape)

    @pl.when(overlap)
    def _compute():
        rows = i0 * bi + jax.lax.broadcasted_iota(jnp.int32, (bi, bj), 0)
        cols = j0 * bj + jax.lax.broadcasted_iota(jnp.int32, (bi, bj), 1)
        m = jnp.logical_and(seg_ref[...] == 0.0,
                            rowcode_ref[...] == colcode_ref[...])
        m = jnp.logical_and(m, rows != cols)
        mf = m.astype(jnp.float32)
        m2x = jax.lax.dot_general(
            mf, f_ref[...], (((1,), (0,)), ((), ())),
            precision=jax.lax.Precision.HIGHEST)
        abig = jax.lax.dot_general(
            z_ref[...], e_ref[...], (((1,), (0,)), ((), ())),
            precision=jax.lax.Precision.HIGHEST)
        outer = abig * zflat_ref[...]
        out_ref[...] = patt + m2x * (outer - patt)


@jax.jit
def kernel(z, seg_matrix, cls_label, batch):
    n, r = z.shape
    bi, bj = _BI, _BJ

    z = z.astype(jnp.float32)
    seg_matrix = seg_matrix.astype(jnp.float32)
    cls_label = cls_label.astype(jnp.int32)
    batch = batch.astype(jnp.int32)

    node_mask = (cls_label != 24) & (cls_label != 25) & (cls_label != 26)
    rowcode = jnp.where(node_mask, batch, -1).astype(jnp.float32)[:, None]
    colcode = jnp.where(node_mask, batch, -3).astype(jnp.float32)[None, :]

    zflat = z.reshape(1, n * r)
    lane = jax.lax.broadcasted_iota(jnp.int32, (1, bj * r), 1)
    patt = (lane % r == 0).astype(jnp.float32)
    e_mat = (jax.lax.broadcasted_iota(jnp.int32, (r, bj * r), 0)
             == lane % r).astype(jnp.float32)
    f_mat = (jax.lax.broadcasted_iota(jnp.int32, (bj, bj * r), 0)
             == lane // r).astype(jnp.float32)

    grid = (n // bi, n // bj)
    out2d = pl.pallas_call(
        functools.partial(_frd_kernel, bi=bi, bj=bj, r=r),
        grid_spec=pltpu.PrefetchScalarGridSpec(
            num_scalar_prefetch=1,
            grid=grid,
            in_specs=[
                pl.BlockSpec((bi, 1), lambda i, j, b: (i, 0)),
                pl.BlockSpec((1, bj), lambda i, j, b: (0, j)),
                pl.BlockSpec((bi, r), lambda i, j, b: (i, 0)),
                pl.BlockSpec((1, bj * r), lambda i, j, b: (0, j)),
                pl.BlockSpec((1, bj * r), lambda i, j, b: (0, 0)),
                pl.BlockSpec((r, bj * r), lambda i, j, b: (0, 0)),
                pl.BlockSpec((bj, bj * r), lambda i, j, b: (0, 0)),
                pl.BlockSpec((bi, bj), lambda i, j, b: (i, j)),
            ],
            out_specs=pl.BlockSpec((bi, bj * r), lambda i, j, b: (i, j)),
        ),
        out_shape=jax.ShapeDtypeStruct((n, n * r), jnp.float32),
    )(batch, rowcode, colcode, z, zflat, patt, e_mat, f_mat, seg_matrix)
    return out2d.reshape(n, n, r)


# speedup vs baseline: 1.2317x; 1.2317x over previous
import functools

import jax
import jax.numpy as jnp
from jax.experimental import pallas as pl
from jax.experimental.pallas import tpu as pltpu

_BI = 128
_BJ = 128


def _frd_kernel(batch_sm, rowcode_ref, colcode_ref, zhi_ref, zlo_ref,
                zflat_ref, patt_ref, e_ref, f_ref, seg_ref, out_ref,
                *, bi, bj, r, nj):
    i0 = pl.program_id(0)
    row_lo = batch_sm[i0 * bi]
    row_hi = batch_sm[i0 * bi + bi - 1]
    patt = patt_ref[...]
    w = bj * r

    for j in range(nj):
        col_lo = batch_sm[j * bj]
        col_hi = batch_sm[j * bj + bj - 1]
        overlap = jnp.logical_and(row_lo <= col_hi, col_lo <= row_hi)

        @pl.when(jnp.logical_not(overlap))
        def _pattern_only(j=j):
            out_ref[:, j * w:(j + 1) * w] = jnp.broadcast_to(patt, (bi, w))

        @pl.when(overlap)
        def _compute(j=j):
            rows = i0 * bi + jax.lax.broadcasted_iota(jnp.int32, (bi, bj), 0)
            cols = j * bj + jax.lax.broadcasted_iota(jnp.int32, (bi, bj), 1)
            m = jnp.logical_and(
                seg_ref[:, j * bj:(j + 1) * bj] == 0.0,
                rowcode_ref[...] == colcode_ref[:, j * bj:(j + 1) * bj])
            m = jnp.logical_and(m, rows != cols)
            mf = m.astype(jnp.bfloat16)
            m2x = jax.lax.dot_general(
                mf, f_ref[...], (((1,), (0,)), ((), ())),
                preferred_element_type=jnp.float32)
            abig = jax.lax.dot_general(
                zhi_ref[...], e_ref[...], (((1,), (0,)), ((), ())),
                preferred_element_type=jnp.float32)
            abig = abig + jax.lax.dot_general(
                zlo_ref[...], e_ref[...], (((1,), (0,)), ((), ())),
                preferred_element_type=jnp.float32)
            outer = abig * zflat_ref[:, j * w:(j + 1) * w]
            out_ref[:, j * w:(j + 1) * w] = patt + m2x * (outer - patt)


@jax.jit
def kernel(z, seg_matrix, cls_label, batch):
    n, r = z.shape
    bi, bj = _BI, _BJ
    nj = n // bj

    z = z.astype(jnp.float32)
    seg_matrix = seg_matrix.astype(jnp.float32)
    cls_label = cls_label.astype(jnp.int32)
    batch = batch.astype(jnp.int32)

    node_mask = (cls_label != 24) & (cls_label != 25) & (cls_label != 26)
    rowcode = jnp.where(node_mask, batch, -1).astype(jnp.float32)[:, None]
    colcode = jnp.where(node_mask, batch, -3).astype(jnp.float32)[None, :]

    z_hi = z.astype(jnp.bfloat16)
    z_lo = (z - z_hi.astype(jnp.float32)).astype(jnp.bfloat16)
    zflat = z.reshape(1, n * r)
    lane = jax.lax.broadcasted_iota(jnp.int32, (1, bj * r), 1)
    patt = (lane % r == 0).astype(jnp.float32)
    e_mat = (jax.lax.broadcasted_iota(jnp.int32, (r, bj * r), 0)
             == lane % r).astype(jnp.bfloat16)
    f_mat = (jax.lax.broadcasted_iota(jnp.int32, (bj, bj * r), 0)
             == lane // r).astype(jnp.bfloat16)

    grid = (n // bi,)
    out2d = pl.pallas_call(
        functools.partial(_frd_kernel, bi=bi, bj=bj, r=r, nj=nj),
        grid_spec=pltpu.PrefetchScalarGridSpec(
            num_scalar_prefetch=1,
            grid=grid,
            in_specs=[
                pl.BlockSpec((bi, 1), lambda i, b: (i, 0)),
                pl.BlockSpec((1, n), lambda i, b: (0, 0)),
                pl.BlockSpec((bi, r), lambda i, b: (i, 0)),
                pl.BlockSpec((bi, r), lambda i, b: (i, 0)),
                pl.BlockSpec((1, n * r), lambda i, b: (0, 0)),
                pl.BlockSpec((1, bj * r), lambda i, b: (0, 0)),
                pl.BlockSpec((r, bj * r), lambda i, b: (0, 0)),
                pl.BlockSpec((bj, bj * r), lambda i, b: (0, 0)),
                pl.BlockSpec((bi, n), lambda i, b: (i, 0)),
            ],
            out_specs=pl.BlockSpec((bi, n * r), lambda i, b: (i, 0)),
        ),
        out_shape=jax.ShapeDtypeStruct((n, n * r), jnp.float32),
    )(batch, rowcode, colcode, z_hi, z_lo, zflat, patt, e_mat, f_mat,
      seg_matrix)
    return out2d.reshape(n, n, r)


# speedup vs baseline: 5.9260x; 4.8112x over previous
import functools

import jax
import jax.numpy as jnp
from jax.experimental import pallas as pl
from jax.experimental.pallas import tpu as pltpu

_BI = 128
_BJ = 128


def _frd_kernel(batch_sm, rowcode_ref, colcode_ref, zcol_ref, zt_ref,
                patt_ref, seg_ref, out_ref, *, bi, bj, r, nj):
    i0 = pl.program_id(0)
    row_lo = batch_sm[i0 * bi]
    row_hi = batch_sm[i0 * bi + bi - 1]

    out_ref[...] = jnp.broadcast_to(patt_ref[...], out_ref.shape)

    zcol = zcol_ref[...]
    for j in range(nj):
        col_lo = batch_sm[j * bj]
        col_hi = batch_sm[j * bj + bj - 1]
        overlap = jnp.logical_and(row_lo <= col_hi, col_lo <= row_hi)

        @pl.when(overlap)
        def _compute(j=j):
            js = slice(j * bj, (j + 1) * bj)
            rows = i0 * bi + jax.lax.broadcasted_iota(jnp.int32, (bi, bj), 0)
            cols = j * bj + jax.lax.broadcasted_iota(jnp.int32, (bi, bj), 1)
            m = jnp.logical_and(seg_ref[:, js] == 0.0,
                                rowcode_ref[...] == colcode_ref[:, js])
            m = jnp.logical_and(m, rows != cols)
            m3 = jnp.broadcast_to(m[:, None, :], (bi, r, bj))
            outer = zcol * zt_ref[:, :, js]
            out_ref[:, :, js] = jnp.where(m3, outer, patt_ref[:, :, js])


@jax.jit
def kernel(z, seg_matrix, cls_label, batch):
    n, r = z.shape
    bi, bj = _BI, _BJ
    nj = n // bj

    z = z.astype(jnp.float32)
    seg_matrix = seg_matrix.astype(jnp.float32)
    cls_label = cls_label.astype(jnp.int32)
    batch = batch.astype(jnp.int32)

    node_mask = (cls_label != 24) & (cls_label != 25) & (cls_label != 26)
    rowcode = jnp.where(node_mask, batch, -1).astype(jnp.float32)[:, None]
    colcode = jnp.where(node_mask, batch, -3).astype(jnp.float32)[None, :]

    zcol = z[:, :, None]
    zt = z.T[None, :, :]
    patt = (jax.lax.broadcasted_iota(jnp.int32, (1, r, n), 1)
            == 0).astype(jnp.float32)

    grid = (n // bi,)
    out3 = pl.pallas_call(
        functools.partial(_frd_kernel, bi=bi, bj=bj, r=r, nj=nj),
        grid_spec=pltpu.PrefetchScalarGridSpec(
            num_scalar_prefetch=1,
            grid=grid,
            in_specs=[
                pl.BlockSpec((bi, 1), lambda i, b: (i, 0)),
                pl.BlockSpec((1, n), lambda i, b: (0, 0)),
                pl.BlockSpec((bi, r, 1), lambda i, b: (i, 0, 0)),
                pl.BlockSpec((1, r, n), lambda i, b: (0, 0, 0)),
                pl.BlockSpec((1, r, n), lambda i, b: (0, 0, 0)),
                pl.BlockSpec((bi, n), lambda i, b: (i, 0)),
            ],
            out_specs=pl.BlockSpec((bi, r, n), lambda i, b: (i, 0, 0)),
        ),
        out_shape=jax.ShapeDtypeStruct((n, r, n), jnp.float32),
    )(batch, rowcode, colcode, zcol, zt, patt, seg_matrix)
    return out3.transpose(0, 2, 1)


# speedup vs baseline: 6.8503x; 1.1560x over previous
import functools

import jax
import jax.numpy as jnp
from jax.experimental import pallas as pl
from jax.experimental.pallas import tpu as pltpu

_BI = 128
_BJ = 128


def _frd_kernel(batch_sm, rowcode_ref, colcode_ref, z_ref, zt_ref,
                patt_ref, seg_ref, out_ref, *, bi, bj, r, nj):
    i0 = pl.program_id(0)
    row_lo = batch_sm[i0 * bi]
    row_hi = batch_sm[i0 * bi + bi - 1]

    out_ref[...] = jnp.broadcast_to(patt_ref[...], out_ref.shape)

    zcol = z_ref[...][:, :, None]
    for j in range(nj):
        col_lo = batch_sm[j * bj]
        col_hi = batch_sm[j * bj + bj - 1]
        overlap = jnp.logical_and(row_lo <= col_hi, col_lo <= row_hi)

        @pl.when(overlap)
        def _compute(j=j):
            js = slice(j * bj, (j + 1) * bj)
            rows = i0 * bi + jax.lax.broadcasted_iota(jnp.int32, (bi, bj), 0)
            cols = j * bj + jax.lax.broadcasted_iota(jnp.int32, (bi, bj), 1)
            m = jnp.logical_and(seg_ref[:, js] == 0.0,
                                rowcode_ref[...] == colcode_ref[:, js])
            m = jnp.logical_and(m, rows != cols)
            m3 = jnp.broadcast_to(m[:, None, :], (bi, r, bj))
            outer = zcol * zt_ref[:, :, js]
            out_ref[:, :, js] = jnp.where(m3, outer, patt_ref[:, :, js])


@jax.jit
def kernel(z, seg_matrix, cls_label, batch):
    n, r = z.shape
    bi, bj = _BI, _BJ
    nj = n // bj

    z = z.astype(jnp.float32)
    seg_matrix = seg_matrix.astype(jnp.float32)
    cls_label = cls_label.astype(jnp.int32)
    batch = batch.astype(jnp.int32)

    node_mask = (cls_label != 24) & (cls_label != 25) & (cls_label != 26)
    rowcode = jnp.where(node_mask, batch, -1).astype(jnp.float32)[:, None]
    colcode = jnp.where(node_mask, batch, -3).astype(jnp.float32)[None, :]

    zt = z.T[None, :, :]
    patt = (jax.lax.broadcasted_iota(jnp.int32, (1, r, n), 1)
            == 0).astype(jnp.float32)

    grid = (n // bi,)
    out3 = pl.pallas_call(
        functools.partial(_frd_kernel, bi=bi, bj=bj, r=r, nj=nj),
        grid_spec=pltpu.PrefetchScalarGridSpec(
            num_scalar_prefetch=1,
            grid=grid,
            in_specs=[
                pl.BlockSpec((bi, 1), lambda i, b: (i, 0)),
                pl.BlockSpec((1, n), lambda i, b: (0, 0)),
                pl.BlockSpec((bi, r), lambda i, b: (i, 0)),
                pl.BlockSpec((1, r, n), lambda i, b: (0, 0, 0)),
                pl.BlockSpec((1, r, n), lambda i, b: (0, 0, 0)),
                pl.BlockSpec((bi, n), lambda i, b: (i, 0)),
            ],
            out_specs=pl.BlockSpec((bi, r, n), lambda i, b: (i, 0, 0)),
        ),
        out_shape=jax.ShapeDtypeStruct((n, r, n), jnp.float32),
    )(batch, rowcode, colcode, z, zt, patt, seg_matrix)
    return out3.transpose(0, 2, 1)


# speedup vs baseline: 7.1239x; 1.0399x over previous
import functools

import jax
import jax.numpy as jnp
from jax.experimental import pallas as pl
from jax.experimental.pallas import tpu as pltpu

_BI = 128
_BJ = 128


def _frd_kernel(batch_sm, rowcode_ref, colcode_ref, z_ref, zt_ref,
                patt_ref, seg_hbm, out_ref, seg_scr, seg_sem,
                *, bi, bj, r, nj):
    i0 = pl.program_id(0)
    row_lo = batch_sm[i0 * bi]
    row_hi = batch_sm[i0 * bi + bi - 1]

    overlaps = []
    for j in range(nj):
        col_lo = batch_sm[j * bj]
        col_hi = batch_sm[j * bj + bj - 1]
        overlaps.append(jnp.logical_and(row_lo <= col_hi, col_lo <= row_hi))

    for j in range(nj):
        @pl.when(overlaps[j])
        def _start(j=j):
            pltpu.make_async_copy(
                seg_hbm.at[pl.ds(i0 * bi, bi), pl.ds(j * bj, bj)],
                seg_scr.at[j], seg_sem.at[j]).start()

    out_ref[...] = jnp.broadcast_to(patt_ref[...], out_ref.shape)

    zcol = z_ref[...][:, :, None]
    for j in range(nj):
        @pl.when(overlaps[j])
        def _compute(j=j):
            pltpu.make_async_copy(
                seg_hbm.at[pl.ds(i0 * bi, bi), pl.ds(j * bj, bj)],
                seg_scr.at[j], seg_sem.at[j]).wait()
            js = slice(j * bj, (j + 1) * bj)
            rows = i0 * bi + jax.lax.broadcasted_iota(jnp.int32, (bi, bj), 0)
            cols = j * bj + jax.lax.broadcasted_iota(jnp.int32, (bi, bj), 1)
            m = jnp.logical_and(seg_scr[j] == 0.0,
                                rowcode_ref[...] == colcode_ref[:, js])
            m = jnp.logical_and(m, rows != cols)
            m3 = jnp.broadcast_to(m[:, None, :], (bi, r, bj))
            outer = zcol * zt_ref[:, :, js]
            out_ref[:, :, js] = jnp.where(m3, outer, patt_ref[:, :, js])


@jax.jit
def kernel(z, seg_matrix, cls_label, batch):
    n, r = z.shape
    bi, bj = _BI, _BJ
    nj = n // bj

    z = z.astype(jnp.float32)
    seg_matrix = seg_matrix.astype(jnp.float32)
    cls_label = cls_label.astype(jnp.int32)
    batch = batch.astype(jnp.int32)

    node_mask = (cls_label != 24) & (cls_label != 25) & (cls_label != 26)
    rowcode = jnp.where(node_mask, batch, -1).astype(jnp.float32)[:, None]
    colcode = jnp.where(node_mask, batch, -3).astype(jnp.float32)[None, :]

    zt = z.T[None, :, :]
    patt = (jax.lax.broadcasted_iota(jnp.int32, (1, r, n), 1)
            == 0).astype(jnp.float32)

    grid = (n // bi,)
    out3 = pl.pallas_call(
        functools.partial(_frd_kernel, bi=bi, bj=bj, r=r, nj=nj),
        grid_spec=pltpu.PrefetchScalarGridSpec(
            num_scalar_prefetch=1,
            grid=grid,
            in_specs=[
                pl.BlockSpec((bi, 1), lambda i, b: (i, 0)),
                pl.BlockSpec((1, n), lambda i, b: (0, 0)),
                pl.BlockSpec((bi, r), lambda i, b: (i, 0)),
                pl.BlockSpec((1, r, n), lambda i, b: (0, 0, 0)),
                pl.BlockSpec((1, r, n), lambda i, b: (0, 0, 0)),
                pl.BlockSpec(memory_space=pltpu.MemorySpace.HBM),
            ],
            out_specs=pl.BlockSpec((bi, r, n), lambda i, b: (i, 0, 0)),
            scratch_shapes=[
                pltpu.VMEM((nj, bi, bj), jnp.float32),
                pltpu.SemaphoreType.DMA((nj,)),
            ],
        ),
        out_shape=jax.ShapeDtypeStruct((n, r, n), jnp.float32),
    )(batch, rowcode, colcode, z, zt, patt, seg_matrix)
    return out3.transpose(0, 2, 1)


# speedup vs baseline: 7.2855x; 1.0227x over previous
import functools

import jax
import jax.numpy as jnp
from jax.experimental import pallas as pl
from jax.experimental.pallas import tpu as pltpu

_BI = 128
_BJ = 128


def _frd_kernel(batch_sm, colcode_ref, z_ref, zt_ref,
                patt_ref, seg_hbm, out_ref, seg_scr, seg_sem,
                *, bi, bj, r, nj):
    i0 = pl.program_id(0)
    row_lo = batch_sm[i0 * bi]
    row_hi = batch_sm[i0 * bi + bi - 1]

    overlaps = []
    for j in range(nj):
        col_lo = batch_sm[j * bj]
        col_hi = batch_sm[j * bj + bj - 1]
        overlaps.append(jnp.logical_and(row_lo <= col_hi, col_lo <= row_hi))

    for j in range(nj):
        @pl.when(overlaps[j])
        def _start(j=j):
            pltpu.make_async_copy(
                seg_hbm.at[pl.ds(i0 * bi, bi), pl.ds(j * bj, bj)],
                seg_scr.at[j], seg_sem.at[j]).start()

    out_ref[...] = jnp.broadcast_to(patt_ref[...], out_ref.shape)

    zcol = z_ref[...][:, :, None]
    rowcode = jnp.transpose(colcode_ref[:, pl.ds(i0 * bi, bi)], (1, 0))
    row_valid = rowcode >= 0.0
    for j in range(nj):
        @pl.when(overlaps[j])
        def _compute(j=j):
            pltpu.make_async_copy(
                seg_hbm.at[pl.ds(i0 * bi, bi), pl.ds(j * bj, bj)],
                seg_scr.at[j], seg_sem.at[j]).wait()
            js = slice(j * bj, (j + 1) * bj)
            rows = i0 * bi + jax.lax.broadcasted_iota(jnp.int32, (bi, bj), 0)
            cols = j * bj + jax.lax.broadcasted_iota(jnp.int32, (bi, bj), 1)
            m = jnp.logical_and(seg_scr[j] == 0.0,
                                rowcode == colcode_ref[:, js])
            m = jnp.logical_and(m, row_valid)
            m = jnp.logical_and(m, rows != cols)
            m3 = jnp.broadcast_to(m[:, None, :], (bi, r, bj))
            outer = zcol * zt_ref[:, :, js]
            out_ref[:, :, js] = jnp.where(m3, outer, patt_ref[:, :, js])


@jax.jit
def kernel(z, seg_matrix, cls_label, batch):
    n, r = z.shape
    bi, bj = _BI, _BJ
    nj = n // bj

    z = z.astype(jnp.float32)
    seg_matrix = seg_matrix.astype(jnp.float32)
    cls_label = cls_label.astype(jnp.int32)
    batch = batch.astype(jnp.int32)

    node_mask = (cls_label != 24) & (cls_label != 25) & (cls_label != 26)
    colcode = jnp.where(node_mask, batch, -3).astype(jnp.float32)[None, :]

    zt = z.T[None, :, :]
    patt = (jax.lax.broadcasted_iota(jnp.int32, (1, r, n), 1)
            == 0).astype(jnp.float32)

    grid = (n // bi,)
    out3 = pl.pallas_call(
        functools.partial(_frd_kernel, bi=bi, bj=bj, r=r, nj=nj),
        grid_spec=pltpu.PrefetchScalarGridSpec(
            num_scalar_prefetch=1,
            grid=grid,
            in_specs=[
                pl.BlockSpec((1, n), lambda i, b: (0, 0)),
                pl.BlockSpec((bi, r), lambda i, b: (i, 0)),
                pl.BlockSpec((1, r, n), lambda i, b: (0, 0, 0)),
                pl.BlockSpec((1, r, n), lambda i, b: (0, 0, 0)),
                pl.BlockSpec(memory_space=pltpu.MemorySpace.HBM),
            ],
            out_specs=pl.BlockSpec((bi, r, n), lambda i, b: (i, 0, 0)),
            scratch_shapes=[
                pltpu.VMEM((nj, bi, bj), jnp.float32),
                pltpu.SemaphoreType.DMA((nj,)),
            ],
        ),
        out_shape=jax.ShapeDtypeStruct((n, r, n), jnp.float32),
    )(batch, colcode, z, zt, patt, seg_matrix)
    return out3.transpose(0, 2, 1)
